# Initial kernel scaffold; baseline (speedup 1.0000x reference)
import jax
import jax.numpy as jnp
from jax.experimental import pallas as pl


def kernel(idx_a1, idx_a2, idx_a3, idx_a4, idx_p1, idx_p2, idx_p3, idx_p4, emb_a1, emb_a2, emb_a3, emb_a4, emb_p1, emb_p2, emb_p3, emb_p4, Wa1_1, ba1_1, Wa2_1, ba2_1, Wa3_1, ba3_1, Wa4_1, ba4_1, Wp1_1, bp1_1, Wa1_2, ba1_2, Wp1_2, bp1_2):
    raise NotImplementedError("write your pallas kernel here")


# speedup vs baseline: 4.6564x; 4.6564x over previous
import jax
import jax.numpy as jnp
from jax import lax
from jax.experimental import pallas as pl

B = 16384
BLK = 2048
NF = 8


def _body(idx_ref, emb_ref, w1_ref, b1_ref, w2_ref, b2_ref, out_ref):
    tables = []
    for n in range(NF):
        e = emb_ref[n]
        h = jnp.dot(e, w1_ref[n], preferred_element_type=jnp.float32)
        h = jnp.maximum(h + b1_ref[n, :][None, :], 0.0)
        t = jnp.dot(h, w2_ref[n], preferred_element_type=jnp.float32)
        t = jnp.maximum(t + b2_ref[n, :][None, :], 0.0)
        tables.append(t)

    iota = lax.broadcasted_iota(jnp.int32, (BLK, 256), 1)
    for c in range(B // BLK):
        idx_c = idx_ref[pl.ds(c * BLK, BLK), :]
        cols = []
        for n in range(NF):
            onehot = (idx_c[:, n][:, None] == iota).astype(jnp.float32)
            cols.append(jnp.dot(onehot, tables[n],
                                preferred_element_type=jnp.float32))
        out_ref[pl.ds(c * BLK, BLK), :] = jnp.concatenate(cols, axis=1)


def kernel(idx_a1, idx_a2, idx_a3, idx_a4, idx_p1, idx_p2, idx_p3, idx_p4,
           emb_a1, emb_a2, emb_a3, emb_a4, emb_p1, emb_p2, emb_p3, emb_p4,
           Wa1_1, ba1_1, Wa2_1, ba2_1, Wa3_1, ba3_1, Wa4_1, ba4_1,
           Wp1_1, bp1_1, Wa1_2, ba1_2, Wp1_2, bp1_2):
    idx = jnp.stack([idx_a1, idx_a2, idx_a3, idx_a4,
                     idx_p1, idx_p2, idx_p3, idx_p4], axis=1)
    emb = jnp.stack([emb_a1, emb_a2, emb_a3, emb_a4,
                     emb_p1, emb_p2, emb_p3, emb_p4])
    w1 = jnp.stack([Wa1_1, Wa2_1, Wa3_1, Wa4_1,
                    Wp1_1, Wp1_1, Wp1_1, Wp1_1])
    b1 = jnp.stack([ba1_1, ba2_1, ba3_1, ba4_1,
                    bp1_1, bp1_1, bp1_1, bp1_1])
    w2 = jnp.stack([Wa1_2, Wa1_2, Wa1_2, Wa1_2,
                    Wp1_2, Wp1_2, Wp1_2, Wp1_2])
    b2 = jnp.stack([ba1_2, ba1_2, ba1_2, ba1_2,
                    bp1_2, bp1_2, bp1_2, bp1_2])

    return pl.pallas_call(
        _body,
        out_shape=jax.ShapeDtypeStruct((B, 16), jnp.float32),
    )(idx, emb, w1, b1, w2, b2)


# speedup vs baseline: 5.9916x; 1.2867x over previous
import functools

import jax
import jax.numpy as jnp
from jax import lax
from jax.experimental import pallas as pl
from jax.experimental.pallas import tpu as pltpu
from jax.experimental.pallas import tpu_sc as plsc

B = 16384
NF = 8

NC = 2
NS = 16
L = 16
NW = NC * NS
RPW = B // NW
OUT_W = 2 * NF


def _table_body(emb_ref, w1_ref, b1_ref, w2_ref, b2_ref, t_ref):
    for n in range(NF):
        e = emb_ref[n]
        h = jnp.dot(e, w1_ref[n], preferred_element_type=jnp.float32)
        h = jnp.maximum(h + b1_ref[n, :][None, :], 0.0)
        t = jnp.dot(h, w2_ref[n], preferred_element_type=jnp.float32)
        t = jnp.maximum(t + b2_ref[n, :][None, :], 0.0)
        t_ref[pl.ds(n * 256, 256), :] = t


_sc_mesh = plsc.VectorSubcoreMesh(core_axis_name="c", subcore_axis_name="s")


@functools.partial(
    pl.kernel,
    mesh=_sc_mesh,
    out_type=jax.ShapeDtypeStruct((B * OUT_W,), jnp.float32),
    scratch_types=[
        pltpu.VMEM((NF * 256 * 2,), jnp.float32),
        pltpu.VMEM((NF, RPW), jnp.int32),
        pltpu.VMEM((RPW * OUT_W,), jnp.float32),
    ],
    compiler_params=pltpu.CompilerParams(needs_layout_passes=False),
)
def _sc_gather(t_hbm, idx_hbm, out_hbm, t_v, idx_v, out_v):
    wid = lax.axis_index("s") * NC + lax.axis_index("c")
    pltpu.sync_copy(t_hbm, t_v)
    pltpu.sync_copy(idx_hbm.at[wid], idx_v)

    lane = lax.iota(jnp.int32, L)
    nchunks = RPW // L
    for n in range(NF):
        pos_n = lane * OUT_W + 2 * n

        def body(c, _, n=n, pos_n=pos_n):
            g2 = (idx_v[n, pl.ds(c * L, L)] + n * 256) * 2
            v0 = plsc.load_gather(t_v, [g2])
            v1 = plsc.load_gather(t_v, [g2 + 1])
            pos = pos_n + c * (L * OUT_W)
            plsc.store_scatter(out_v, [pos], v0)
            plsc.store_scatter(out_v, [pos + 1], v1)
            return 0

        lax.fori_loop(0, nchunks, body, 0)

    pltpu.sync_copy(out_v, out_hbm.at[pl.ds(wid * (RPW * OUT_W), RPW * OUT_W)])


def kernel(idx_a1, idx_a2, idx_a3, idx_a4, idx_p1, idx_p2, idx_p3, idx_p4,
           emb_a1, emb_a2, emb_a3, emb_a4, emb_p1, emb_p2, emb_p3, emb_p4,
           Wa1_1, ba1_1, Wa2_1, ba2_1, Wa3_1, ba3_1, Wa4_1, ba4_1,
           Wp1_1, bp1_1, Wa1_2, ba1_2, Wp1_2, bp1_2):
    emb = jnp.stack([emb_a1, emb_a2, emb_a3, emb_a4,
                     emb_p1, emb_p2, emb_p3, emb_p4])
    w1 = jnp.stack([Wa1_1, Wa2_1, Wa3_1, Wa4_1,
                    Wp1_1, Wp1_1, Wp1_1, Wp1_1])
    b1 = jnp.stack([ba1_1, ba2_1, ba3_1, ba4_1,
                    bp1_1, bp1_1, bp1_1, bp1_1])
    w2 = jnp.stack([Wa1_2, Wa1_2, Wa1_2, Wa1_2,
                    Wp1_2, Wp1_2, Wp1_2, Wp1_2])
    b2 = jnp.stack([ba1_2, ba1_2, ba1_2, ba1_2,
                    bp1_2, bp1_2, bp1_2, bp1_2])

    t = pl.pallas_call(
        _table_body,
        out_shape=jax.ShapeDtypeStruct((NF * 256, 2), jnp.float32),
    )(emb, w1, b1, w2, b2)

    idx = jnp.stack([idx_a1, idx_a2, idx_a3, idx_a4,
                     idx_p1, idx_p2, idx_p3, idx_p4])
    idx_t = idx.reshape(NF, NW, RPW).transpose(1, 0, 2)

    out_flat = _sc_gather(t.reshape(NF * 256 * 2), idx_t)
    return out_flat.reshape(B, OUT_W)
